# baseline (speedup 1.0000x reference)
import functools

import jax
import jax.numpy as jnp
from jax import lax
from jax.experimental import pallas as pl
from jax.experimental.pallas import tpu as pltpu
from jax.experimental.pallas import tpu_sc as plsc

_V = 1000
_D = 1000
_B, _L = 1024, 50

_NC, _NS = 2, 16
_NW = _NC * _NS
_DV = 32
_LASTV = _V - _DV
_NG = _B // 16


def _make_gather():
    mesh = plsc.VectorSubcoreMesh(core_axis_name="c", subcore_axis_name="s")

    @functools.partial(
        pl.kernel,
        out_type=jax.ShapeDtypeStruct((_L, _D, _B), jnp.float32),
        mesh=mesh,
        scratch_types=[
            pltpu.VMEM((_DV * _V,), jnp.float32),
            pltpu.VMEM((2, _B), jnp.int32),
            pltpu.VMEM((2, _DV, _B), jnp.float32),
            pltpu.SemaphoreType.DMA,
            [pltpu.SemaphoreType.DMA] * 2,
            [pltpu.SemaphoreType.DMA] * 2,
        ],
        compiler_params=pltpu.CompilerParams(needs_layout_passes=False),
    )
    def gather_kernel(idxt_hbm, tablet_hbm, out_hbm, slab, idxb, ob,
                      slabsem, isems, osems):
        wid = lax.axis_index("s") * _NC + lax.axis_index("c")
        last = wid == _NW - 1
        v0 = jnp.where(last, _LASTV, wid * _DV)

        pltpu.async_copy(
            tablet_hbm.at[pl.ds(v0 * _V, _DV * _V)], slab, slabsem).wait()

        def i_desc(l, p):
            return pltpu.make_async_copy(
                idxt_hbm.at[l], idxb.at[p], isems[p])

        def o_desc(l, p):
            def mk(lo, n):
                return pltpu.make_async_copy(
                    ob.at[p, pl.ds(lo, n)],
                    out_hbm.at[l, pl.ds(v0 + lo, n)], osems[p])
            return mk

        def o_start(l, p):
            mk = o_desc(l, p)

            @pl.when(last)
            def _():
                mk(_DV - 8, 8).start()

            @pl.when(jnp.logical_not(last))
            def _():
                mk(0, _DV).start()

        def o_wait(l, p):
            mk = o_desc(l, p)

            @pl.when(last)
            def _():
                mk(_DV - 8, 8).wait()

            @pl.when(jnp.logical_not(last))
            def _():
                mk(0, _DV).wait()

        def compute(l, p):
            @plsc.parallel_loop(0, _NG, unroll=2)
            def group(g):
                col = g * 16
                vidx = idxb[p, pl.ds(col, 16)]
                for v in range(_DV):
                    vals = plsc.load_gather(
                        slab.at[pl.ds(v * _V, _V)], [vidx])
                    ob[p, v, pl.ds(col, 16)] = vals

        i_desc(0, 0).start()

        def pair(h, carry):
            l0 = 2 * h
            for p in range(2):
                l = l0 + p
                i_desc(l, p).wait()

                @pl.when(l + 1 < _L)
                def _():
                    i_desc(l + 1, 1 - p).start()

                @pl.when(l >= 2)
                def _():
                    o_wait(l - 2, p)

                compute(l, p)
                o_start(l, p)
            return carry

        lax.fori_loop(0, _L // 2, pair, 0)
        o_wait(_L - 2, 0)
        o_wait(_L - 1, 1)

    return gather_kernel


_gather = _make_gather()


@jax.jit
def kernel(idx, logits_table):
    out_t = _gather(idx.astype(jnp.int32).T, logits_table.T.reshape(-1))
    return out_t.transpose(2, 0, 1)


# speedup vs baseline: 1.0192x; 1.0192x over previous
import functools

import jax
import jax.numpy as jnp
from jax import lax
from jax.experimental import pallas as pl
from jax.experimental.pallas import tpu as pltpu
from jax.experimental.pallas import tpu_sc as plsc

_V = 1000
_D = 1000
_B, _L = 1024, 50

_NC, _NS = 2, 16
_NW = _NC * _NS
_DV = 32
_LASTV = _V - _DV
_NG = _B // 16


def _make_gather():
    mesh = plsc.VectorSubcoreMesh(core_axis_name="c", subcore_axis_name="s")

    @functools.partial(
        pl.kernel,
        out_type=jax.ShapeDtypeStruct((_L, _D, _B), jnp.float32),
        mesh=mesh,
        scratch_types=[
            pltpu.VMEM((_DV * _V,), jnp.float32),
            pltpu.VMEM((2, _B), jnp.int32),
            pltpu.VMEM((2, _DV, _B), jnp.float32),
            pltpu.SemaphoreType.DMA,
            [pltpu.SemaphoreType.DMA] * 2,
            [pltpu.SemaphoreType.DMA] * 2,
        ],
        compiler_params=pltpu.CompilerParams(needs_layout_passes=False),
    )
    def gather_kernel(idxt_hbm, tablet_hbm, out_hbm, slab, idxb, ob,
                      slabsem, isems, osems):
        wid = lax.axis_index("s") * _NC + lax.axis_index("c")
        last = wid == _NW - 1
        v0 = jnp.where(last, _LASTV, wid * _DV)

        pltpu.async_copy(
            tablet_hbm.at[pl.ds(v0 * _V, _DV * _V)], slab, slabsem).wait()

        def i_desc(l, p):
            return pltpu.make_async_copy(
                idxt_hbm.at[pl.ds(l * _B, _B)], idxb.at[p], isems[p])

        def o_desc(l, p):
            def mk(lo, n):
                return pltpu.make_async_copy(
                    ob.at[p, pl.ds(lo, n)],
                    out_hbm.at[l, pl.ds(v0 + lo, n)], osems[p])
            return mk

        def o_start(l, p):
            mk = o_desc(l, p)

            @pl.when(last)
            def _():
                mk(_DV - 8, 8).start()

            @pl.when(jnp.logical_not(last))
            def _():
                mk(0, _DV).start()

        def o_wait(l, p):
            mk = o_desc(l, p)

            @pl.when(last)
            def _():
                mk(_DV - 8, 8).wait()

            @pl.when(jnp.logical_not(last))
            def _():
                mk(0, _DV).wait()

        def compute(l, p):
            @plsc.parallel_loop(0, _NG, unroll=4)
            def group(g):
                col = g * 16
                vidx = idxb[p, pl.ds(col, 16)]
                for v in range(_DV):
                    vals = plsc.load_gather(
                        slab.at[pl.ds(v * _V, _V)], [vidx])
                    ob[p, v, pl.ds(col, 16)] = vals

        i_desc(0, 0).start()

        def pair(h, carry):
            l0 = 2 * h
            for p in range(2):
                l = l0 + p
                i_desc(l, p).wait()

                @pl.when(l + 1 < _L)
                def _():
                    i_desc(l + 1, 1 - p).start()

                @pl.when(l >= 2)
                def _():
                    o_wait(l - 2, p)

                compute(l, p)
                o_start(l, p)
            return carry

        lax.fori_loop(0, _L // 2, pair, 0)
        o_wait(_L - 2, 0)
        o_wait(_L - 1, 1)

    return gather_kernel


_gather = _make_gather()


@jax.jit
def kernel(idx, logits_table):
    idxt_flat = idx.astype(jnp.int32).T.reshape(-1)
    out_t = _gather(idxt_flat, logits_table.T.reshape(-1))
    return out_t.transpose(2, 0, 1)


# speedup vs baseline: 1.0844x; 1.0639x over previous
import functools

import jax
import jax.numpy as jnp
from jax import lax
from jax.experimental import pallas as pl
from jax.experimental.pallas import tpu as pltpu
from jax.experimental.pallas import tpu_sc as plsc

_V = 1000
_D = 1000
_B, _L = 1024, 50

_NC, _NS = 2, 16
_NW = _NC * _NS
_DV = 32
_LASTV = _V - _DV
_NG = _B // 16


def _make_gather():
    mesh = plsc.VectorSubcoreMesh(core_axis_name="c", subcore_axis_name="s")

    @functools.partial(
        pl.kernel,
        out_type=jax.ShapeDtypeStruct((_L, _D, _B), jnp.float32),
        mesh=mesh,
        scratch_types=[
            pltpu.VMEM((_DV * _V,), jnp.float32),
            pltpu.VMEM((2, 8, _B), jnp.int32),
            pltpu.VMEM((2, _DV, _B), jnp.float32),
            pltpu.SemaphoreType.DMA,
            pltpu.SemaphoreType.DMA((2,)),
            [pltpu.SemaphoreType.DMA] * 2,
        ],
        compiler_params=pltpu.CompilerParams(needs_layout_passes=False),
    )
    def gather_kernel(idxt_hbm, tablet_hbm, out_hbm, slab, idxb, ob,
                      slabsem, isems_arr, osems):
        wid = lax.axis_index("s") * _NC + lax.axis_index("c")
        last = wid == _NW - 1
        v0 = jnp.where(last, _LASTV, wid * _DV)

        pltpu.async_copy(
            tablet_hbm.at[pl.ds(v0 * _V, _DV * _V)], slab, slabsem).wait()

        _NBLK = (_L + 7) // 8

        def i_start(blk):
            p = lax.rem(blk, 2)

            @pl.when(blk < _NBLK - 1)
            def _():
                pltpu.make_async_copy(
                    idxt_hbm.at[pl.ds(blk * 8, 8)], idxb.at[p],
                    isems_arr.at[p]).start()

            @pl.when(blk == _NBLK - 1)
            def _():
                pltpu.make_async_copy(
                    idxt_hbm.at[pl.ds(blk * 8, 2)], idxb.at[p, pl.ds(0, 2)],
                    isems_arr.at[p]).start()

        def i_wait(blk):
            p = lax.rem(blk, 2)

            @pl.when(blk < _NBLK - 1)
            def _():
                pltpu.make_async_copy(
                    idxt_hbm.at[pl.ds(blk * 8, 8)], idxb.at[p],
                    isems_arr.at[p]).wait()

            @pl.when(blk == _NBLK - 1)
            def _():
                pltpu.make_async_copy(
                    idxt_hbm.at[pl.ds(blk * 8, 2)], idxb.at[p, pl.ds(0, 2)],
                    isems_arr.at[p]).wait()

        def o_desc(l, p):
            def mk(lo, n):
                return pltpu.make_async_copy(
                    ob.at[p, pl.ds(lo, n)],
                    out_hbm.at[l, pl.ds(v0 + lo, n)], osems[p])
            return mk

        def o_start(l, p):
            mk = o_desc(l, p)

            @pl.when(last)
            def _():
                mk(_DV - 8, 8).start()

            @pl.when(jnp.logical_not(last))
            def _():
                mk(0, _DV).start()

        def o_wait(l, p):
            mk = o_desc(l, p)

            @pl.when(last)
            def _():
                mk(_DV - 8, 8).wait()

            @pl.when(jnp.logical_not(last))
            def _():
                mk(0, _DV).wait()

        def compute(l, p):
            islot = lax.rem(lax.div(l, 8), 2)
            lr = lax.rem(l, 8)

            @plsc.parallel_loop(0, _NG, unroll=2)
            def group(g):
                col = g * 16
                vidx = idxb[islot, lr, pl.ds(col, 16)]
                for v in range(_DV):
                    vals = plsc.load_gather(
                        slab.at[pl.ds(v * _V, _V)], [vidx])
                    ob[p, v, pl.ds(col, 16)] = vals

        i_start(0)

        def pair(h, carry):
            l0 = 2 * h
            for p in range(2):
                l = l0 + p

                @pl.when(lax.rem(l, 8) == 0)
                def _():
                    blk = lax.div(l, 8)

                    @pl.when(blk + 1 < _NBLK)
                    def _():
                        i_start(blk + 1)
                    i_wait(blk)

                @pl.when(l >= 2)
                def _():
                    o_wait(l - 2, p)

                compute(l, p)
                o_start(l, p)
            return carry

        lax.fori_loop(0, _L // 2, pair, 0)
        o_wait(_L - 2, 0)
        o_wait(_L - 1, 1)

    return gather_kernel


_gather = _make_gather()


@jax.jit
def kernel(idx, logits_table):
    out_t = _gather(idx.astype(jnp.int32).T, logits_table.T.reshape(-1))
    return out_t.transpose(2, 0, 1)
